# Initial kernel scaffold; baseline (speedup 1.0000x reference)
import jax
import jax.numpy as jnp
from jax.experimental import pallas as pl


def kernel(hidden_4d, sparsity, gate_up_proj):
    raise NotImplementedError("write your pallas kernel here")


# speedup vs baseline: 6.8391x; 6.8391x over previous
import jax
import jax.numpy as jnp
from jax.experimental import pallas as pl
from jax.experimental.pallas import tpu as pltpu

T = 4096
H = 2048
E = 8
O2 = 4096
OC = 4
CH = O2 // OC


def _body(sp_ref, hid_ref, w_ref, out_ref, sh_ref):
    e = pl.program_id(0)
    oc = pl.program_id(1)

    @pl.when((e == 0) & (oc == 0))
    def _init():
        out_ref[...] = jnp.zeros_like(out_ref)
        sh_ref[...] = jnp.dot(sp_ref[...], hid_ref[...],
                              preferred_element_type=jnp.float32)

    wsum = jnp.sum(w_ref[0], axis=-1)
    s_e = sh_ref[pl.ds(e, 1), :]
    out_ref[...] += jnp.sum(s_e[0] * wsum).reshape(1, 1)


def kernel(hidden_4d, sparsity, gate_up_proj):
    hidden = hidden_4d.reshape(T, H)
    sp = sparsity.reshape(E, T)
    w = gate_up_proj.reshape(E, H, O2)
    out = pl.pallas_call(
        _body,
        grid=(E, OC),
        in_specs=[
            pl.BlockSpec((E, T), lambda e, oc: (0, 0)),
            pl.BlockSpec((T, H), lambda e, oc: (0, 0)),
            pl.BlockSpec((1, H, CH), lambda e, oc: (e, 0, oc)),
        ],
        out_specs=pl.BlockSpec((1, 1), lambda e, oc: (0, 0)),
        out_shape=jax.ShapeDtypeStruct((1, 1), jnp.float32),
        scratch_shapes=[pltpu.VMEM((E, H), jnp.float32)],
    )(sp, hidden, w)
    return out[0, 0]
